# Initial kernel scaffold; baseline (speedup 1.0000x reference)
import functools

import jax
import jax.numpy as jnp
from jax import lax
from jax.experimental import pallas as pl
from jax.experimental.pallas import tpu as pltpu
from jax.experimental.pallas import tpu_sc as plsc

B, S, D, DFF, E, K = 2, 2048, 1024, 4096, 8, 2
T = B * S
A = T * K
TILE = 512
PAD_N = A + E * TILE
NT = PAD_N // TILE
DBLK = 2048
NDB = DFF // DBLK

NC, NS = 2, 16
NW = NC * NS

GBLK = 1024


def _gate_body(x_ref, wg_ref, idx_ref, w_ref):
    logits = lax.dot_general(x_ref[...], wg_ref[...],
                             (((1,), (1,)), ((), ())),
                             preferred_element_type=jnp.float32)
    neg = jnp.full((GBLK, 1), -jnp.inf, jnp.float32)
    m1, i1 = neg, jnp.zeros((GBLK, 1), jnp.int32)
    for e in range(E):
        v = logits[:, e:e + 1]
        upd = v > m1
        m1 = jnp.where(upd, v, m1)
        i1 = jnp.where(upd, e, i1)
    m2, i2 = neg, jnp.zeros((GBLK, 1), jnp.int32)
    for e in range(E):
        v = logits[:, e:e + 1]
        upd = jnp.logical_and(v > m2, i1 != e)
        m2 = jnp.where(upd, v, m2)
        i2 = jnp.where(upd, e, i2)
    e2 = jnp.exp(m2 - m1)
    w1 = 1.0 / (1.0 + e2)
    idx_ref[...] = jnp.concatenate([i1, i2], axis=1)
    w_ref[...] = jnp.concatenate([w1, 1.0 - w1], axis=1)


def _gate(x2d, Wg):
    return pl.pallas_call(
        _gate_body,
        grid=(T // GBLK,),
        in_specs=[
            pl.BlockSpec((GBLK, D), lambda i: (i, 0)),
            pl.BlockSpec((E, D), lambda i: (0, 0)),
        ],
        out_specs=[
            pl.BlockSpec((GBLK, K), lambda i: (i, 0)),
            pl.BlockSpec((GBLK, K), lambda i: (i, 0)),
        ],
        out_shape=[
            jax.ShapeDtypeStruct((T, K), jnp.int32),
            jax.ShapeDtypeStruct((T, K), jnp.float32),
        ],
    )(x2d, Wg)


def _dispatch(topi, topw):
    ids = topi.reshape(-1)
    wts = topw.reshape(-1)
    order = jnp.argsort(ids, stable=True).astype(jnp.int32)
    sid = jnp.take(ids, order)
    counts = jnp.zeros((E,), jnp.int32).at[ids].add(1)
    pc = ((counts + TILE - 1) // TILE) * TILE
    pcc = jnp.cumsum(pc)
    poff = pcc - pc
    occ = jnp.cumsum(counts)
    off = occ - counts
    ar = jnp.arange(A, dtype=jnp.int32)
    dest = jnp.take(poff, sid) + (ar - jnp.take(off, sid))
    tok_sorted = jnp.take(ar // K, order)
    gidx = jnp.zeros((PAD_N,), jnp.int32).at[dest].set(tok_sorted)
    gw = jnp.zeros((PAD_N,), jnp.float32).at[dest].set(jnp.take(wts, order))
    pos = jnp.zeros((A,), jnp.int32).at[order].set(dest)
    p0, p1 = pos[0::2], pos[1::2]
    tile_start = jnp.arange(NT, dtype=jnp.int32) * TILE
    tile_e = jnp.minimum(
        jnp.searchsorted(pcc, tile_start, side="right").astype(jnp.int32),
        E - 1)
    tile_act = (tile_start < pcc[E - 1]).astype(jnp.int32)
    return gidx, gw, p0, p1, tile_e, tile_act


_G_CHUNK = 64
_G_PER_W = PAD_N // NW


def _gather_body(tab_hbm, idx_hbm, out_hbm, idx_v, rows_v, sem):
    wid = lax.axis_index("s") * NC + lax.axis_index("c")
    base = wid * _G_PER_W

    def chunk(ci, carry):
        off = base + ci * _G_CHUNK
        pltpu.sync_copy(idx_hbm.at[pl.ds(off, _G_CHUNK)], idx_v)
        pltpu.async_copy(tab_hbm.at[idx_v], rows_v, sem).wait()
        pltpu.sync_copy(rows_v, out_hbm.at[pl.ds(off, _G_CHUNK)])
        return carry

    lax.fori_loop(0, _G_PER_W // _G_CHUNK, chunk, 0)


_sc_gather = functools.partial(
    pl.kernel,
    out_type=jax.ShapeDtypeStruct((PAD_N, D), jnp.float32),
    mesh=plsc.VectorSubcoreMesh(core_axis_name="c", subcore_axis_name="s"),
    scratch_types=[
        pltpu.VMEM((_G_CHUNK,), jnp.int32),
        pltpu.VMEM((_G_CHUNK, D), jnp.float32),
        pltpu.SemaphoreType.DMA,
    ],
)(_gather_body)


def _ffn_body(te_ref, ta_ref, xg_ref, w1_ref, w2_ref, w3_ref,
              b1_ref, b2_ref, b3_ref, gw_ref, out_ref, acc_ref):
    i = pl.program_id(0)
    j = pl.program_id(1)
    act = ta_ref[i] == 1

    @pl.when(act)
    def _():
        x = xg_ref[...].astype(jnp.bfloat16)
        a = lax.dot_general(x, w1_ref[0], (((1,), (1,)), ((), ())),
                            preferred_element_type=jnp.float32) + b1_ref[0]
        b = lax.dot_general(x, w2_ref[0], (((1,), (1,)), ((), ())),
                            preferred_element_type=jnp.float32) + b2_ref[0]
        h = (a * jax.nn.sigmoid(a) * b).astype(jnp.bfloat16)
        y = lax.dot_general(h, w3_ref[0], (((1,), (1,)), ((), ())),
                            preferred_element_type=jnp.float32)

        @pl.when(j == 0)
        def _():
            acc_ref[...] = y

        @pl.when(j > 0)
        def _():
            acc_ref[...] += y

    @pl.when(j == NDB - 1)
    def _():
        @pl.when(act)
        def _():
            out_ref[...] = (acc_ref[...] + b3_ref[0]) * gw_ref[0]

        @pl.when(jnp.logical_not(act))
        def _():
            out_ref[...] = jnp.zeros_like(out_ref)


def _ffn(xg, W1b, W2b, W3b, b1r, b2r, b3r, gw3, tile_e, tile_act):
    grid_spec = pltpu.PrefetchScalarGridSpec(
        num_scalar_prefetch=2,
        grid=(NT, NDB),
        in_specs=[
            pl.BlockSpec((TILE, D), lambda i, j, te, ta: (i, 0)),
            pl.BlockSpec((1, DBLK, D), lambda i, j, te, ta: (te[i], j, 0)),
            pl.BlockSpec((1, DBLK, D), lambda i, j, te, ta: (te[i], j, 0)),
            pl.BlockSpec((1, D, DBLK), lambda i, j, te, ta: (te[i], 0, j)),
            pl.BlockSpec((1, 1, DBLK), lambda i, j, te, ta: (te[i], 0, j)),
            pl.BlockSpec((1, 1, DBLK), lambda i, j, te, ta: (te[i], 0, j)),
            pl.BlockSpec((1, 1, D), lambda i, j, te, ta: (te[i], 0, 0)),
            pl.BlockSpec((1, TILE, 1), lambda i, j, te, ta: (i, 0, 0)),
        ],
        out_specs=pl.BlockSpec((TILE, D), lambda i, j, te, ta: (i, 0)),
        scratch_shapes=[pltpu.VMEM((TILE, D), jnp.float32)],
    )
    return pl.pallas_call(
        _ffn_body,
        grid_spec=grid_spec,
        out_shape=jax.ShapeDtypeStruct((PAD_N, D), jnp.float32),
    )(tile_e, tile_act, xg, W1b, W2b, W3b, b1r, b2r, b3r, gw3)


_C_CHUNK = 32
_C_PER_W = T // NW


def _combine_body(yw_hbm, p0_hbm, p1_hbm, out_hbm, i0_v, i1_v, r0_v, r1_v, sem):
    wid = lax.axis_index("s") * NC + lax.axis_index("c")
    base = wid * _C_PER_W

    def chunk(ci, carry):
        off = base + ci * _C_CHUNK
        pltpu.sync_copy(p0_hbm.at[pl.ds(off, _C_CHUNK)], i0_v)
        pltpu.sync_copy(p1_hbm.at[pl.ds(off, _C_CHUNK)], i1_v)
        cp0 = pltpu.async_copy(yw_hbm.at[i0_v], r0_v, sem)
        cp1 = pltpu.async_copy(yw_hbm.at[i1_v], r1_v, sem)
        cp0.wait()
        cp1.wait()

        def row(ri, c2):
            def col(cj, c3):
                sl = pl.ds(cj * 16, 16)
                r0_v[ri, sl] = r0_v[ri, sl] + r1_v[ri, sl]
                return c3
            return lax.fori_loop(0, D // 16, col, c2)

        lax.fori_loop(0, _C_CHUNK, row, 0)
        pltpu.sync_copy(r0_v, out_hbm.at[pl.ds(off, _C_CHUNK)])
        return carry

    lax.fori_loop(0, _C_PER_W // _C_CHUNK, chunk, 0)


_sc_combine = functools.partial(
    pl.kernel,
    out_type=jax.ShapeDtypeStruct((T, D), jnp.float32),
    mesh=plsc.VectorSubcoreMesh(core_axis_name="c", subcore_axis_name="s"),
    scratch_types=[
        pltpu.VMEM((_C_CHUNK,), jnp.int32),
        pltpu.VMEM((_C_CHUNK,), jnp.int32),
        pltpu.VMEM((_C_CHUNK, D), jnp.float32),
        pltpu.VMEM((_C_CHUNK, D), jnp.float32),
        pltpu.SemaphoreType.DMA,
    ],
)(_combine_body)


def kernel(x, W1, b1, W2, b2, W3, b3, Wg):
    x2d = x.reshape(T, D)
    topi, topw = _gate(x2d, Wg)
    gidx, gw, p0, p1, tile_e, tile_act = _dispatch(topi, topw)
    xg = _sc_gather(x2d, gidx)
    yw = _ffn(xg,
              W1.astype(jnp.bfloat16),
              W2.astype(jnp.bfloat16),
              W3.astype(jnp.bfloat16),
              b1.reshape(E, 1, DFF), b2.reshape(E, 1, DFF),
              b3.reshape(E, 1, D),
              gw.reshape(NT, TILE, 1), tile_e, tile_act)
    out2d = _sc_combine(yw, p0, p1)
    return out2d.reshape(B, S, D)


# speedup vs baseline: 1.2336x; 1.2336x over previous
import functools

import jax
import jax.numpy as jnp
from jax import lax
from jax.experimental import pallas as pl
from jax.experimental.pallas import tpu as pltpu
from jax.experimental.pallas import tpu_sc as plsc

B, S, D, DFF, E, K = 2, 2048, 1024, 4096, 8, 2
T = B * S
A = T * K
TILE = 512
PAD_N = A + E * TILE
NT = PAD_N // TILE
DBLK = 2048
NDB = DFF // DBLK

NC, NS = 2, 16
NW = NC * NS

GBLK = 1024


def _gate_body(x_ref, wg_ref, idx_ref, w_ref):
    logits = lax.dot_general(x_ref[...], wg_ref[...],
                             (((1,), (1,)), ((), ())),
                             preferred_element_type=jnp.float32)
    neg = jnp.full((GBLK, 1), -jnp.inf, jnp.float32)
    m1, i1 = neg, jnp.zeros((GBLK, 1), jnp.int32)
    for e in range(E):
        v = logits[:, e:e + 1]
        upd = v > m1
        m1 = jnp.where(upd, v, m1)
        i1 = jnp.where(upd, e, i1)
    m2, i2 = neg, jnp.zeros((GBLK, 1), jnp.int32)
    for e in range(E):
        v = logits[:, e:e + 1]
        upd = jnp.logical_and(v > m2, i1 != e)
        m2 = jnp.where(upd, v, m2)
        i2 = jnp.where(upd, e, i2)
    e2 = jnp.exp(m2 - m1)
    w1 = 1.0 / (1.0 + e2)
    idx_ref[...] = jnp.concatenate([i1, i2], axis=1)
    w_ref[...] = jnp.concatenate([w1, 1.0 - w1], axis=1)


def _gate(x2d, Wg):
    return pl.pallas_call(
        _gate_body,
        grid=(T // GBLK,),
        in_specs=[
            pl.BlockSpec((GBLK, D), lambda i: (i, 0)),
            pl.BlockSpec((E, D), lambda i: (0, 0)),
        ],
        out_specs=[
            pl.BlockSpec((GBLK, K), lambda i: (i, 0)),
            pl.BlockSpec((GBLK, K), lambda i: (i, 0)),
        ],
        out_shape=[
            jax.ShapeDtypeStruct((T, K), jnp.int32),
            jax.ShapeDtypeStruct((T, K), jnp.float32),
        ],
    )(x2d, Wg)


def _dispatch(topi, topw):
    ids = topi.reshape(-1)
    wts = topw.reshape(-1)
    order = jnp.argsort(ids, stable=True).astype(jnp.int32)
    sid = jnp.take(ids, order)
    counts = jnp.zeros((E,), jnp.int32).at[ids].add(1)
    pc = ((counts + TILE - 1) // TILE) * TILE
    pcc = jnp.cumsum(pc)
    poff = pcc - pc
    occ = jnp.cumsum(counts)
    off = occ - counts
    ar = jnp.arange(A, dtype=jnp.int32)
    dest = jnp.take(poff, sid) + (ar - jnp.take(off, sid))
    tok_sorted = jnp.take(ar // K, order)
    gidx = jnp.zeros((PAD_N,), jnp.int32).at[dest].set(tok_sorted)
    gw = jnp.zeros((PAD_N,), jnp.float32).at[dest].set(jnp.take(wts, order))
    pos = jnp.zeros((A,), jnp.int32).at[order].set(dest)
    p0, p1 = pos[0::2], pos[1::2]
    tile_start = jnp.arange(NT, dtype=jnp.int32) * TILE
    tile_e = jnp.minimum(
        jnp.searchsorted(pcc, tile_start, side="right").astype(jnp.int32),
        E - 1)
    tile_act = (tile_start < pcc[E - 1]).astype(jnp.int32)
    return gidx, gw, p0, p1, tile_e, tile_act


_G_CHUNK = 64
_G_PER_W = PAD_N // NW


def _gather_body(tab_hbm, idx_hbm, out_hbm, idx_v, rows_v, sem):
    wid = lax.axis_index("s") * NC + lax.axis_index("c")
    base = wid * _G_PER_W

    def chunk(ci, carry):
        off = base + ci * _G_CHUNK
        pltpu.sync_copy(idx_hbm.at[pl.ds(off, _G_CHUNK)], idx_v)
        pltpu.async_copy(tab_hbm.at[idx_v], rows_v, sem).wait()
        pltpu.sync_copy(rows_v, out_hbm.at[pl.ds(off, _G_CHUNK)])
        return carry

    lax.fori_loop(0, _G_PER_W // _G_CHUNK, chunk, 0)


@functools.cache
def _sc_gather_kernel():
    return pl.kernel(
        _gather_body,
        out_type=jax.ShapeDtypeStruct((PAD_N, D), jnp.float32),
        mesh=plsc.VectorSubcoreMesh(core_axis_name="c", subcore_axis_name="s",
                                    num_cores=NC, num_subcores=NS),
        scratch_types=[
            pltpu.VMEM((_G_CHUNK,), jnp.int32),
            pltpu.VMEM((_G_CHUNK, D), jnp.float32),
            pltpu.SemaphoreType.DMA,
        ],
    )


def _sc_gather(tab, idx):
    return _sc_gather_kernel()(tab, idx)


def _ffn_body(te_ref, ta_ref, xg_ref, w1_ref, w2_ref, w3_ref,
              b1_ref, b2_ref, b3_ref, gw_ref, out_ref, acc_ref):
    i = pl.program_id(0)
    j = pl.program_id(1)
    act = ta_ref[i] == 1

    @pl.when(act)
    def _():
        x = xg_ref[...].astype(jnp.bfloat16)
        a = lax.dot_general(x, w1_ref[0], (((1,), (1,)), ((), ())),
                            preferred_element_type=jnp.float32) + b1_ref[0]
        b = lax.dot_general(x, w2_ref[0], (((1,), (1,)), ((), ())),
                            preferred_element_type=jnp.float32) + b2_ref[0]
        h = (a * jax.nn.sigmoid(a) * b).astype(jnp.bfloat16)
        y = lax.dot_general(h, w3_ref[0], (((1,), (1,)), ((), ())),
                            preferred_element_type=jnp.float32)

        @pl.when(j == 0)
        def _():
            acc_ref[...] = y

        @pl.when(j > 0)
        def _():
            acc_ref[...] += y

    @pl.when(j == NDB - 1)
    def _():
        @pl.when(act)
        def _():
            out_ref[...] = (acc_ref[...] + b3_ref[0]) * gw_ref[0]

        @pl.when(jnp.logical_not(act))
        def _():
            out_ref[...] = jnp.zeros_like(out_ref)


def _ffn(xg, W1b, W2b, W3b, b1r, b2r, b3r, gw3, tile_e, tile_act):
    grid_spec = pltpu.PrefetchScalarGridSpec(
        num_scalar_prefetch=2,
        grid=(NT, NDB),
        in_specs=[
            pl.BlockSpec((TILE, D), lambda i, j, te, ta: (i, 0)),
            pl.BlockSpec((1, DBLK, D), lambda i, j, te, ta: (te[i], j, 0)),
            pl.BlockSpec((1, DBLK, D), lambda i, j, te, ta: (te[i], j, 0)),
            pl.BlockSpec((1, D, DBLK), lambda i, j, te, ta: (te[i], 0, j)),
            pl.BlockSpec((1, 1, DBLK), lambda i, j, te, ta: (te[i], 0, j)),
            pl.BlockSpec((1, 1, DBLK), lambda i, j, te, ta: (te[i], 0, j)),
            pl.BlockSpec((1, 1, D), lambda i, j, te, ta: (te[i], 0, 0)),
            pl.BlockSpec((1, TILE, 1), lambda i, j, te, ta: (i, 0, 0)),
        ],
        out_specs=pl.BlockSpec((TILE, D), lambda i, j, te, ta: (i, 0)),
        scratch_shapes=[pltpu.VMEM((TILE, D), jnp.float32)],
    )
    return pl.pallas_call(
        _ffn_body,
        grid_spec=grid_spec,
        out_shape=jax.ShapeDtypeStruct((PAD_N, D), jnp.float32),
    )(tile_e, tile_act, xg, W1b, W2b, W3b, b1r, b2r, b3r, gw3)


_C_CHUNK = 32
_C_PER_W = T // NW


def _combine_body(yw_hbm, p0_hbm, p1_hbm, out_hbm, i0_v, i1_v, r0_v, r1_v, sem):
    wid = lax.axis_index("s") * NC + lax.axis_index("c")
    base = wid * _C_PER_W

    def chunk(ci, carry):
        off = base + ci * _C_CHUNK
        pltpu.sync_copy(p0_hbm.at[pl.ds(off, _C_CHUNK)], i0_v)
        pltpu.sync_copy(p1_hbm.at[pl.ds(off, _C_CHUNK)], i1_v)
        cp0 = pltpu.async_copy(yw_hbm.at[i0_v], r0_v, sem)
        cp1 = pltpu.async_copy(yw_hbm.at[i1_v], r1_v, sem)
        cp0.wait()
        cp1.wait()

        def row(ri, c2):
            def col(cj, c3):
                sl = pl.ds(cj * 16, 16)
                r0_v[ri, sl] = r0_v[ri, sl] + r1_v[ri, sl]
                return c3
            return lax.fori_loop(0, D // 16, col, c2)

        lax.fori_loop(0, _C_CHUNK, row, 0)
        pltpu.sync_copy(r0_v, out_hbm.at[pl.ds(off, _C_CHUNK)])
        return carry

    lax.fori_loop(0, _C_PER_W // _C_CHUNK, chunk, 0)


@functools.cache
def _sc_combine_kernel():
    return pl.kernel(
        _combine_body,
        out_type=jax.ShapeDtypeStruct((T, D), jnp.float32),
        mesh=plsc.VectorSubcoreMesh(core_axis_name="c", subcore_axis_name="s",
                                    num_cores=NC, num_subcores=NS),
        scratch_types=[
            pltpu.VMEM((_C_CHUNK,), jnp.int32),
            pltpu.VMEM((_C_CHUNK,), jnp.int32),
            pltpu.VMEM((_C_CHUNK, D), jnp.float32),
            pltpu.VMEM((_C_CHUNK, D), jnp.float32),
            pltpu.SemaphoreType.DMA,
        ],
    )


def _sc_combine(yw, p0, p1):
    return _sc_combine_kernel()(yw, p0, p1)


def kernel(x, W1, b1, W2, b2, W3, b3, Wg):
    x2d = x.reshape(T, D)
    topi, topw = _gate(x2d, Wg)
    gidx, gw, p0, p1, tile_e, tile_act = _dispatch(topi, topw)
    xg = _sc_gather(x2d, gidx)
    yw = _ffn(xg,
              W1.astype(jnp.bfloat16),
              W2.astype(jnp.bfloat16),
              W3.astype(jnp.bfloat16),
              b1.reshape(E, 1, DFF), b2.reshape(E, 1, DFF),
              b3.reshape(E, 1, D),
              gw.reshape(NT, TILE, 1), tile_e, tile_act)
    out2d = _sc_combine(yw, p0, p1)
    return out2d.reshape(B, S, D)


# speedup vs baseline: 1.2443x; 1.0087x over previous
import functools

import jax
import jax.numpy as jnp
from jax import lax
from jax.experimental import pallas as pl
from jax.experimental.pallas import tpu as pltpu
from jax.experimental.pallas import tpu_sc as plsc

B, S, D, DFF, E, K = 2, 2048, 1024, 4096, 8, 2
T = B * S
A = T * K
TILE = 512
PAD_N = A + E * TILE
NT = PAD_N // TILE
DBLK = 2048
NDB = DFF // DBLK

NC, NS = 2, 16
NW = NC * NS

GBLK = 1024


def _gate_body(x_ref, wg_ref, idx_ref, w_ref):
    logits = lax.dot_general(x_ref[...], wg_ref[...],
                             (((1,), (1,)), ((), ())),
                             preferred_element_type=jnp.float32)
    neg = jnp.full((GBLK, 1), -jnp.inf, jnp.float32)
    m1, i1 = neg, jnp.zeros((GBLK, 1), jnp.int32)
    for e in range(E):
        v = logits[:, e:e + 1]
        upd = v > m1
        m1 = jnp.where(upd, v, m1)
        i1 = jnp.where(upd, e, i1)
    m2, i2 = neg, jnp.zeros((GBLK, 1), jnp.int32)
    for e in range(E):
        v = logits[:, e:e + 1]
        upd = jnp.logical_and(v > m2, i1 != e)
        m2 = jnp.where(upd, v, m2)
        i2 = jnp.where(upd, e, i2)
    e2 = jnp.exp(m2 - m1)
    w1 = 1.0 / (1.0 + e2)
    idx_ref[...] = jnp.concatenate([i1, i2], axis=1)
    w_ref[...] = jnp.concatenate([w1, 1.0 - w1], axis=1)


def _gate(x2d, Wg):
    return pl.pallas_call(
        _gate_body,
        grid=(T // GBLK,),
        in_specs=[
            pl.BlockSpec((GBLK, D), lambda i: (i, 0)),
            pl.BlockSpec((E, D), lambda i: (0, 0)),
        ],
        out_specs=[
            pl.BlockSpec((GBLK, K), lambda i: (i, 0)),
            pl.BlockSpec((GBLK, K), lambda i: (i, 0)),
        ],
        out_shape=[
            jax.ShapeDtypeStruct((T, K), jnp.int32),
            jax.ShapeDtypeStruct((T, K), jnp.float32),
        ],
    )(x2d, Wg)


def _dispatch(topi, topw):
    ids = topi.reshape(-1)
    wts = topw.reshape(-1)
    order = jnp.argsort(ids, stable=True).astype(jnp.int32)
    sid = jnp.take(ids, order)
    counts = jnp.zeros((E,), jnp.int32).at[ids].add(1)
    pc = ((counts + TILE - 1) // TILE) * TILE
    pcc = jnp.cumsum(pc)
    poff = pcc - pc
    occ = jnp.cumsum(counts)
    off = occ - counts
    ar = jnp.arange(A, dtype=jnp.int32)
    dest = jnp.take(poff, sid) + (ar - jnp.take(off, sid))
    tok_sorted = jnp.take(ar // K, order)
    gidx = jnp.zeros((PAD_N,), jnp.int32).at[dest].set(tok_sorted)
    gw = jnp.zeros((PAD_N,), jnp.float32).at[dest].set(jnp.take(wts, order))
    pos = jnp.zeros((A,), jnp.int32).at[order].set(dest)
    p0, p1 = pos[0::2], pos[1::2]
    tile_start = jnp.arange(NT, dtype=jnp.int32) * TILE
    tile_e = jnp.minimum(
        jnp.searchsorted(pcc, tile_start, side="right").astype(jnp.int32),
        E - 1)
    tile_act = (tile_start < pcc[E - 1]).astype(jnp.int32)
    return gidx, gw, p0, p1, tile_e, tile_act


_G_CHUNK = 32


def _gather_body(n_rows, tab_hbm, idx_hbm, out_hbm,
                 i0, i1, r0, r1, sem):
    per_w = n_rows // NW
    nch = per_w // _G_CHUNK
    wid = lax.axis_index("s") * NC + lax.axis_index("c")
    base = wid * per_w
    bufs = ((i0, r0), (i1, r1))

    pltpu.sync_copy(idx_hbm.at[pl.ds(base, _G_CHUNK)], i0)
    cp = pltpu.async_copy(tab_hbm.at[i0], r0, sem)
    for c in range(nch):
        _, rb = bufs[c % 2]
        if c + 1 < nch:
            inx, rnx = bufs[(c + 1) % 2]
            pltpu.sync_copy(
                idx_hbm.at[pl.ds(base + (c + 1) * _G_CHUNK, _G_CHUNK)], inx)
            cpn = pltpu.async_copy(tab_hbm.at[inx], rnx, sem)
        cp.wait()
        pltpu.sync_copy(rb, out_hbm.at[pl.ds(base + c * _G_CHUNK, _G_CHUNK)])
        if c + 1 < nch:
            cp = cpn


@functools.cache
def _sc_gather_kernel(n_rows):
    return pl.kernel(
        functools.partial(_gather_body, n_rows),
        out_type=jax.ShapeDtypeStruct((n_rows, D), jnp.float32),
        mesh=plsc.VectorSubcoreMesh(core_axis_name="c", subcore_axis_name="s",
                                    num_cores=NC, num_subcores=NS),
        scratch_types=[
            pltpu.VMEM((_G_CHUNK,), jnp.int32),
            pltpu.VMEM((_G_CHUNK,), jnp.int32),
            pltpu.VMEM((_G_CHUNK, D), jnp.float32),
            pltpu.VMEM((_G_CHUNK, D), jnp.float32),
            pltpu.SemaphoreType.DMA,
        ],
    )


def _sc_gather(tab, idx, n_rows):
    return _sc_gather_kernel(n_rows)(tab, idx)


def _ffn_body(te_ref, ta_ref, xg_ref, w1_ref, w2_ref, w3_ref,
              b1_ref, b2_ref, b3_ref, gw_ref, out_ref, acc_ref):
    i = pl.program_id(0)
    j = pl.program_id(1)
    act = ta_ref[i] == 1

    @pl.when(act)
    def _():
        x = xg_ref[...].astype(jnp.bfloat16)
        a = lax.dot_general(x, w1_ref[0], (((1,), (1,)), ((), ())),
                            preferred_element_type=jnp.float32) + b1_ref[0]
        b = lax.dot_general(x, w2_ref[0], (((1,), (1,)), ((), ())),
                            preferred_element_type=jnp.float32) + b2_ref[0]
        h = (a * jax.nn.sigmoid(a) * b).astype(jnp.bfloat16)
        y = lax.dot_general(h, w3_ref[0], (((1,), (1,)), ((), ())),
                            preferred_element_type=jnp.float32)

        @pl.when(j == 0)
        def _():
            acc_ref[...] = y

        @pl.when(j > 0)
        def _():
            acc_ref[...] += y

    @pl.when(j == NDB - 1)
    def _():
        @pl.when(act)
        def _():
            out_ref[...] = (acc_ref[...] + b3_ref[0]) * gw_ref[0]

        @pl.when(jnp.logical_not(act))
        def _():
            out_ref[...] = jnp.zeros_like(out_ref)


def _ffn(xg, W1b, W2b, W3b, b1r, b2r, b3r, gw3, tile_e, tile_act):
    grid_spec = pltpu.PrefetchScalarGridSpec(
        num_scalar_prefetch=2,
        grid=(NT, NDB),
        in_specs=[
            pl.BlockSpec((TILE, D), lambda i, j, te, ta: (i, 0)),
            pl.BlockSpec((1, DBLK, D), lambda i, j, te, ta: (te[i], j, 0)),
            pl.BlockSpec((1, DBLK, D), lambda i, j, te, ta: (te[i], j, 0)),
            pl.BlockSpec((1, D, DBLK), lambda i, j, te, ta: (te[i], 0, j)),
            pl.BlockSpec((1, 1, DBLK), lambda i, j, te, ta: (te[i], 0, j)),
            pl.BlockSpec((1, 1, DBLK), lambda i, j, te, ta: (te[i], 0, j)),
            pl.BlockSpec((1, 1, D), lambda i, j, te, ta: (te[i], 0, 0)),
            pl.BlockSpec((1, TILE, 1), lambda i, j, te, ta: (i, 0, 0)),
        ],
        out_specs=pl.BlockSpec((TILE, D), lambda i, j, te, ta: (i, 0)),
        scratch_shapes=[pltpu.VMEM((TILE, D), jnp.float32)],
    )
    return pl.pallas_call(
        _ffn_body,
        grid_spec=grid_spec,
        out_shape=jax.ShapeDtypeStruct((PAD_N, D), jnp.float32),
    )(tile_e, tile_act, xg, W1b, W2b, W3b, b1r, b2r, b3r, gw3)


_ADD_BLK = 512


def _add_body(a_ref, b_ref, o_ref):
    o_ref[...] = a_ref[...] + b_ref[...]


def _pair_add(zz):
    return pl.pallas_call(
        _add_body,
        grid=(T // _ADD_BLK,),
        in_specs=[
            pl.BlockSpec((_ADD_BLK, D), lambda i: (i, 0)),
            pl.BlockSpec((_ADD_BLK, D), lambda i: (T // _ADD_BLK + i, 0)),
        ],
        out_specs=pl.BlockSpec((_ADD_BLK, D), lambda i: (i, 0)),
        out_shape=jax.ShapeDtypeStruct((T, D), jnp.float32),
    )(zz, zz)


def kernel(x, W1, b1, W2, b2, W3, b3, Wg):
    x2d = x.reshape(T, D)
    topi, topw = _gate(x2d, Wg)
    gidx, gw, p0, p1, tile_e, tile_act = _dispatch(topi, topw)
    xg = _sc_gather(x2d, gidx, PAD_N)
    yw = _ffn(xg,
              W1.astype(jnp.bfloat16),
              W2.astype(jnp.bfloat16),
              W3.astype(jnp.bfloat16),
              b1.reshape(E, 1, DFF), b2.reshape(E, 1, DFF),
              b3.reshape(E, 1, D),
              gw.reshape(NT, TILE, 1), tile_e, tile_act)
    zz = _sc_gather(yw, jnp.concatenate([p0, p1]), 2 * T)
    return _pair_add(zz).reshape(B, S, D)


# speedup vs baseline: 1.2446x; 1.0002x over previous
import functools

import jax
import jax.numpy as jnp
from jax import lax
from jax.experimental import pallas as pl
from jax.experimental.pallas import tpu as pltpu
from jax.experimental.pallas import tpu_sc as plsc

B, S, D, DFF, E, K = 2, 2048, 1024, 4096, 8, 2
T = B * S
A = T * K
TILE = 512
PAD_N = A + E * TILE
NT = PAD_N // TILE
DBLK = 2048
NDB = DFF // DBLK

NC, NS = 2, 16
NW = NC * NS

GBLK = 1024


def _gate_body(x_ref, wg_ref, idx_ref, w_ref):
    logits = lax.dot_general(x_ref[...], wg_ref[...],
                             (((1,), (1,)), ((), ())),
                             preferred_element_type=jnp.float32)
    neg = jnp.full((GBLK, 1), -jnp.inf, jnp.float32)
    m1, i1 = neg, jnp.zeros((GBLK, 1), jnp.int32)
    for e in range(E):
        v = logits[:, e:e + 1]
        upd = v > m1
        m1 = jnp.where(upd, v, m1)
        i1 = jnp.where(upd, e, i1)
    m2, i2 = neg, jnp.zeros((GBLK, 1), jnp.int32)
    for e in range(E):
        v = logits[:, e:e + 1]
        upd = jnp.logical_and(v > m2, i1 != e)
        m2 = jnp.where(upd, v, m2)
        i2 = jnp.where(upd, e, i2)
    e2 = jnp.exp(m2 - m1)
    w1 = 1.0 / (1.0 + e2)
    idx_ref[...] = jnp.concatenate([i1, i2], axis=1)
    w_ref[...] = jnp.concatenate([w1, 1.0 - w1], axis=1)


def _gate(x2d, Wg):
    return pl.pallas_call(
        _gate_body,
        grid=(T // GBLK,),
        in_specs=[
            pl.BlockSpec((GBLK, D), lambda i: (i, 0)),
            pl.BlockSpec((E, D), lambda i: (0, 0)),
        ],
        out_specs=[
            pl.BlockSpec((GBLK, K), lambda i: (i, 0)),
            pl.BlockSpec((GBLK, K), lambda i: (i, 0)),
        ],
        out_shape=[
            jax.ShapeDtypeStruct((T, K), jnp.int32),
            jax.ShapeDtypeStruct((T, K), jnp.float32),
        ],
    )(x2d, Wg)


def _dispatch(topi, topw):
    ids = topi.reshape(-1)
    wts = topw.reshape(-1)
    order = jnp.argsort(ids, stable=True).astype(jnp.int32)
    sid = jnp.take(ids, order)
    counts = jnp.zeros((E,), jnp.int32).at[ids].add(1)
    pc = ((counts + TILE - 1) // TILE) * TILE
    pcc = jnp.cumsum(pc)
    poff = pcc - pc
    occ = jnp.cumsum(counts)
    off = occ - counts
    ar = jnp.arange(A, dtype=jnp.int32)
    dest = jnp.take(poff, sid) + (ar - jnp.take(off, sid))
    tok_sorted = jnp.take(ar // K, order)
    gidx = jnp.zeros((PAD_N,), jnp.int32).at[dest].set(tok_sorted)
    gw = jnp.zeros((PAD_N,), jnp.float32).at[dest].set(jnp.take(wts, order))
    pos = jnp.zeros((A,), jnp.int32).at[order].set(dest)
    p0, p1 = pos[0::2], pos[1::2]
    tile_start = jnp.arange(NT, dtype=jnp.int32) * TILE
    tile_e = jnp.minimum(
        jnp.searchsorted(pcc, tile_start, side="right").astype(jnp.int32),
        E - 1)
    tile_act = (tile_start < pcc[E - 1]).astype(jnp.int32)
    return gidx, gw, p0, p1, tile_e, tile_act


_G_CHUNK = 32


def _gather_body(n_rows, tab_hbm, idx_hbm, out_hbm,
                 i0, i1, r0, r1, sem):
    per_w = n_rows // NW
    nch = per_w // _G_CHUNK
    wid = lax.axis_index("s") * NC + lax.axis_index("c")
    base = wid * per_w
    bufs = ((i0, r0), (i1, r1))

    pltpu.sync_copy(idx_hbm.at[pl.ds(base, _G_CHUNK)], i0)
    cp = pltpu.async_copy(tab_hbm.at[i0], r0, sem)
    for c in range(nch):
        _, rb = bufs[c % 2]
        if c + 1 < nch:
            inx, rnx = bufs[(c + 1) % 2]
            pltpu.sync_copy(
                idx_hbm.at[pl.ds(base + (c + 1) * _G_CHUNK, _G_CHUNK)], inx)
            cpn = pltpu.async_copy(tab_hbm.at[inx], rnx, sem)
        cp.wait()
        pltpu.sync_copy(rb, out_hbm.at[pl.ds(base + c * _G_CHUNK, _G_CHUNK)])
        if c + 1 < nch:
            cp = cpn


@functools.cache
def _sc_gather_kernel(n_rows):
    return pl.kernel(
        functools.partial(_gather_body, n_rows),
        name=f"sc_row_gather_{n_rows}",
        out_type=jax.ShapeDtypeStruct((n_rows, D), jnp.float32),
        mesh=plsc.VectorSubcoreMesh(core_axis_name="c", subcore_axis_name="s",
                                    num_cores=NC, num_subcores=NS),
        scratch_types=[
            pltpu.VMEM((_G_CHUNK,), jnp.int32),
            pltpu.VMEM((_G_CHUNK,), jnp.int32),
            pltpu.VMEM((_G_CHUNK, D), jnp.float32),
            pltpu.VMEM((_G_CHUNK, D), jnp.float32),
            pltpu.SemaphoreType.DMA,
        ],
    )


def _sc_gather(tab, idx, n_rows):
    return _sc_gather_kernel(n_rows)(tab, idx)


def _ffn_body(te_ref, ta_ref, xg_ref, w1_ref, w2_ref, w3_ref,
              b1_ref, b2_ref, b3_ref, gw_ref, out_ref, acc_ref):
    i = pl.program_id(0)
    j = pl.program_id(1)
    act = ta_ref[i] == 1

    @pl.when(act)
    def _():
        x = xg_ref[...].astype(jnp.bfloat16)
        a = lax.dot_general(x, w1_ref[0], (((1,), (1,)), ((), ())),
                            preferred_element_type=jnp.float32) + b1_ref[0]
        b = lax.dot_general(x, w2_ref[0], (((1,), (1,)), ((), ())),
                            preferred_element_type=jnp.float32) + b2_ref[0]
        h = (a * jax.nn.sigmoid(a) * b).astype(jnp.bfloat16)
        y = lax.dot_general(h, w3_ref[0], (((1,), (1,)), ((), ())),
                            preferred_element_type=jnp.float32)

        @pl.when(j == 0)
        def _():
            acc_ref[...] = y

        @pl.when(j > 0)
        def _():
            acc_ref[...] += y

    @pl.when(j == NDB - 1)
    def _():
        @pl.when(act)
        def _():
            out_ref[...] = (acc_ref[...] + b3_ref[0]) * gw_ref[0]

        @pl.when(jnp.logical_not(act))
        def _():
            out_ref[...] = jnp.zeros_like(out_ref)


def _ffn(xg, W1b, W2b, W3b, b1r, b2r, b3r, gw3, tile_e, tile_act):
    grid_spec = pltpu.PrefetchScalarGridSpec(
        num_scalar_prefetch=2,
        grid=(NT, NDB),
        in_specs=[
            pl.BlockSpec((TILE, D), lambda i, j, te, ta: (i, 0)),
            pl.BlockSpec((1, DBLK, D), lambda i, j, te, ta: (te[i], j, 0)),
            pl.BlockSpec((1, DBLK, D), lambda i, j, te, ta: (te[i], j, 0)),
            pl.BlockSpec((1, D, DBLK), lambda i, j, te, ta: (te[i], 0, j)),
            pl.BlockSpec((1, 1, DBLK), lambda i, j, te, ta: (te[i], 0, j)),
            pl.BlockSpec((1, 1, DBLK), lambda i, j, te, ta: (te[i], 0, j)),
            pl.BlockSpec((1, 1, D), lambda i, j, te, ta: (te[i], 0, 0)),
            pl.BlockSpec((1, TILE, 1), lambda i, j, te, ta: (i, 0, 0)),
        ],
        out_specs=pl.BlockSpec((TILE, D), lambda i, j, te, ta: (i, 0)),
        scratch_shapes=[pltpu.VMEM((TILE, D), jnp.float32)],
    )
    return pl.pallas_call(
        _ffn_body,
        grid_spec=grid_spec,
        out_shape=jax.ShapeDtypeStruct((PAD_N, D), jnp.float32),
    )(tile_e, tile_act, xg, W1b, W2b, W3b, b1r, b2r, b3r, gw3)


_ADD_BLK = 512


def _add_body(a_ref, b_ref, o_ref):
    o_ref[...] = a_ref[...] + b_ref[...]


def _pair_add(zz):
    return pl.pallas_call(
        _add_body,
        grid=(T // _ADD_BLK,),
        in_specs=[
            pl.BlockSpec((_ADD_BLK, D), lambda i: (i, 0)),
            pl.BlockSpec((_ADD_BLK, D), lambda i: (T // _ADD_BLK + i, 0)),
        ],
        out_specs=pl.BlockSpec((_ADD_BLK, D), lambda i: (i, 0)),
        out_shape=jax.ShapeDtypeStruct((T, D), jnp.float32),
    )(zz, zz)


def kernel(x, W1, b1, W2, b2, W3, b3, Wg):
    x2d = x.reshape(T, D)
    topi, topw = _gate(x2d, Wg)
    gidx, gw, p0, p1, tile_e, tile_act = _dispatch(topi, topw)
    xg = _sc_gather(x2d, gidx, PAD_N)
    yw = _ffn(xg,
              W1.astype(jnp.bfloat16),
              W2.astype(jnp.bfloat16),
              W3.astype(jnp.bfloat16),
              b1.reshape(E, 1, DFF), b2.reshape(E, 1, DFF),
              b3.reshape(E, 1, D),
              gw.reshape(NT, TILE, 1), tile_e, tile_act)
    zz = _sc_gather(yw, jnp.concatenate([p0, p1]), 2 * T)
    return _pair_add(zz).reshape(B, S, D)


# speedup vs baseline: 1.5699x; 1.2614x over previous
import functools

import jax
import jax.numpy as jnp
from jax import lax
from jax.experimental import pallas as pl
from jax.experimental.pallas import tpu as pltpu
from jax.experimental.pallas import tpu_sc as plsc

B, S, D, DFF, E, K = 2, 2048, 1024, 4096, 8, 2
T = B * S
A = T * K
TILE = 512
PAD_N = A + E * TILE
NT = PAD_N // TILE
DBLK = 2048
NDB = DFF // DBLK

NC, NS = 2, 16
NW = NC * NS

GBLK = 1024


def _gate_body(x_ref, wg_ref, idx_ref, w_ref):
    logits = lax.dot_general(x_ref[...], wg_ref[...],
                             (((1,), (1,)), ((), ())),
                             preferred_element_type=jnp.float32)
    neg = jnp.full((GBLK, 1), -jnp.inf, jnp.float32)
    m1, i1 = neg, jnp.zeros((GBLK, 1), jnp.int32)
    for e in range(E):
        v = logits[:, e:e + 1]
        upd = v > m1
        m1 = jnp.where(upd, v, m1)
        i1 = jnp.where(upd, e, i1)
    m2, i2 = neg, jnp.zeros((GBLK, 1), jnp.int32)
    for e in range(E):
        v = logits[:, e:e + 1]
        upd = jnp.logical_and(v > m2, i1 != e)
        m2 = jnp.where(upd, v, m2)
        i2 = jnp.where(upd, e, i2)
    e2 = jnp.exp(m2 - m1)
    w1 = 1.0 / (1.0 + e2)
    idx_ref[...] = jnp.concatenate([i1, i2], axis=1)
    w_ref[...] = jnp.concatenate([w1, 1.0 - w1], axis=1)


def _gate(x2d, Wg):
    return pl.pallas_call(
        _gate_body,
        grid=(T // GBLK,),
        in_specs=[
            pl.BlockSpec((GBLK, D), lambda i: (i, 0)),
            pl.BlockSpec((E, D), lambda i: (0, 0)),
        ],
        out_specs=[
            pl.BlockSpec((GBLK, K), lambda i: (i, 0)),
            pl.BlockSpec((GBLK, K), lambda i: (i, 0)),
        ],
        out_shape=[
            jax.ShapeDtypeStruct((T, K), jnp.int32),
            jax.ShapeDtypeStruct((T, K), jnp.float32),
        ],
    )(x2d, Wg)


def _dispatch(topi, topw):
    ids = topi.reshape(-1)
    wts = topw.reshape(-1)
    order = jnp.argsort(ids, stable=True).astype(jnp.int32)
    sid = jnp.take(ids, order)
    counts = jnp.zeros((E,), jnp.int32).at[ids].add(1)
    pc = ((counts + TILE - 1) // TILE) * TILE
    pcc = jnp.cumsum(pc)
    poff = pcc - pc
    occ = jnp.cumsum(counts)
    off = occ - counts
    ar = jnp.arange(A, dtype=jnp.int32)
    dest = jnp.take(poff, sid) + (ar - jnp.take(off, sid))
    tok_sorted = jnp.take(ar // K, order)
    gidx = (jnp.arange(PAD_N, dtype=jnp.int32) % T).at[dest].set(tok_sorted)
    gw = jnp.zeros((PAD_N,), jnp.float32).at[dest].set(jnp.take(wts, order))
    pos = jnp.zeros((A,), jnp.int32).at[order].set(dest)
    p0, p1 = pos[0::2], pos[1::2]
    tile_start = jnp.arange(NT, dtype=jnp.int32) * TILE
    tile_e = jnp.minimum(
        jnp.searchsorted(pcc, tile_start, side="right").astype(jnp.int32),
        E - 1)
    tile_act = (tile_start < pcc[E - 1]).astype(jnp.int32)
    return gidx, gw, p0, p1, tile_e, tile_act


_G_CHUNK = 32


def _gather_body(n_rows, tab_hbm, idx_hbm, out_hbm,
                 i0, i1, r0, r1, sem):
    per_w = n_rows // NW
    nch = per_w // _G_CHUNK
    wid = lax.axis_index("s") * NC + lax.axis_index("c")
    base = wid * per_w
    bufs = ((i0, r0), (i1, r1))

    pltpu.sync_copy(idx_hbm.at[pl.ds(base, _G_CHUNK)], i0)
    cp = pltpu.async_copy(tab_hbm.at[i0], r0, sem)
    for c in range(nch):
        _, rb = bufs[c % 2]
        if c + 1 < nch:
            inx, rnx = bufs[(c + 1) % 2]
            pltpu.sync_copy(
                idx_hbm.at[pl.ds(base + (c + 1) * _G_CHUNK, _G_CHUNK)], inx)
            cpn = pltpu.async_copy(tab_hbm.at[inx], rnx, sem)
        cp.wait()
        pltpu.sync_copy(rb, out_hbm.at[pl.ds(base + c * _G_CHUNK, _G_CHUNK)])
        if c + 1 < nch:
            cp = cpn


@functools.cache
def _sc_gather_kernel(n_rows):
    return pl.kernel(
        functools.partial(_gather_body, n_rows),
        name=f"sc_row_gather_{n_rows}",
        out_type=jax.ShapeDtypeStruct((n_rows, D), jnp.float32),
        mesh=plsc.VectorSubcoreMesh(core_axis_name="c", subcore_axis_name="s",
                                    num_cores=NC, num_subcores=NS),
        scratch_types=[
            pltpu.VMEM((_G_CHUNK,), jnp.int32),
            pltpu.VMEM((_G_CHUNK,), jnp.int32),
            pltpu.VMEM((_G_CHUNK, D), jnp.float32),
            pltpu.VMEM((_G_CHUNK, D), jnp.float32),
            pltpu.SemaphoreType.DMA,
        ],
    )


def _sc_gather(tab, idx, n_rows):
    return _sc_gather_kernel(n_rows)(tab, idx)


def _ffn_body(te_ref, ta_ref, xg_ref, w1_ref, w2_ref, w3_ref,
              b1_ref, b2_ref, b3_ref, gw_ref, out_ref, acc_ref):
    i = pl.program_id(0)
    j = pl.program_id(1)
    act = ta_ref[i] == 1

    @pl.when(act)
    def _():
        x = xg_ref[...].astype(jnp.bfloat16)
        a = lax.dot_general(x, w1_ref[0], (((1,), (1,)), ((), ())),
                            preferred_element_type=jnp.float32) + b1_ref[0]
        b = lax.dot_general(x, w2_ref[0], (((1,), (1,)), ((), ())),
                            preferred_element_type=jnp.float32) + b2_ref[0]
        h = (a * jax.nn.sigmoid(a) * b).astype(jnp.bfloat16)
        y = lax.dot_general(h, w3_ref[0], (((1,), (1,)), ((), ())),
                            preferred_element_type=jnp.float32)

        @pl.when(j == 0)
        def _():
            acc_ref[...] = y

        @pl.when(j > 0)
        def _():
            acc_ref[...] += y

    @pl.when(j == NDB - 1)
    def _():
        @pl.when(act)
        def _():
            out_ref[...] = (acc_ref[...] + b3_ref[0]) * gw_ref[0]

        @pl.when(jnp.logical_not(act))
        def _():
            out_ref[...] = jnp.zeros_like(out_ref)


def _ffn(xg, W1b, W2b, W3b, b1r, b2r, b3r, gw3, tile_e, tile_act):
    grid_spec = pltpu.PrefetchScalarGridSpec(
        num_scalar_prefetch=2,
        grid=(NT, NDB),
        in_specs=[
            pl.BlockSpec((TILE, D), lambda i, j, te, ta: (i, 0)),
            pl.BlockSpec((1, DBLK, D), lambda i, j, te, ta: (te[i], j, 0)),
            pl.BlockSpec((1, DBLK, D), lambda i, j, te, ta: (te[i], j, 0)),
            pl.BlockSpec((1, D, DBLK), lambda i, j, te, ta: (te[i], 0, j)),
            pl.BlockSpec((1, 1, DBLK), lambda i, j, te, ta: (te[i], 0, j)),
            pl.BlockSpec((1, 1, DBLK), lambda i, j, te, ta: (te[i], 0, j)),
            pl.BlockSpec((1, 1, D), lambda i, j, te, ta: (te[i], 0, 0)),
            pl.BlockSpec((1, TILE, 1), lambda i, j, te, ta: (i, 0, 0)),
        ],
        out_specs=pl.BlockSpec((TILE, D), lambda i, j, te, ta: (i, 0)),
        scratch_shapes=[pltpu.VMEM((TILE, D), jnp.float32)],
    )
    return pl.pallas_call(
        _ffn_body,
        grid_spec=grid_spec,
        out_shape=jax.ShapeDtypeStruct((PAD_N, D), jnp.float32),
    )(tile_e, tile_act, xg, W1b, W2b, W3b, b1r, b2r, b3r, gw3)


_ADD_BLK = 512


def _add_body(a_ref, b_ref, o_ref):
    o_ref[...] = a_ref[...] + b_ref[...]


def _pair_add(zz):
    return pl.pallas_call(
        _add_body,
        grid=(T // _ADD_BLK,),
        in_specs=[
            pl.BlockSpec((_ADD_BLK, D), lambda i: (i, 0)),
            pl.BlockSpec((_ADD_BLK, D), lambda i: (T // _ADD_BLK + i, 0)),
        ],
        out_specs=pl.BlockSpec((_ADD_BLK, D), lambda i: (i, 0)),
        out_shape=jax.ShapeDtypeStruct((T, D), jnp.float32),
    )(zz, zz)


def kernel(x, W1, b1, W2, b2, W3, b3, Wg):
    x2d = x.reshape(T, D)
    topi, topw = _gate(x2d, Wg)
    gidx, gw, p0, p1, tile_e, tile_act = _dispatch(topi, topw)
    xg = _sc_gather(x2d, gidx, PAD_N)
    yw = _ffn(xg,
              W1.astype(jnp.bfloat16),
              W2.astype(jnp.bfloat16),
              W3.astype(jnp.bfloat16),
              b1.reshape(E, 1, DFF), b2.reshape(E, 1, DFF),
              b3.reshape(E, 1, D),
              gw.reshape(NT, TILE, 1), tile_e, tile_act)
    zz = _sc_gather(yw, jnp.concatenate([p0, p1]), 2 * T)
    return _pair_add(zz).reshape(B, S, D)


# speedup vs baseline: 1.6994x; 1.0825x over previous
import functools

import jax
import jax.numpy as jnp
from jax import lax
from jax.experimental import pallas as pl
from jax.experimental.pallas import tpu as pltpu
from jax.experimental.pallas import tpu_sc as plsc

B, S, D, DFF, E, K = 2, 2048, 1024, 4096, 8, 2
T = B * S
A = T * K
TILE = 512
PAD_N = A + E * TILE
NT = PAD_N // TILE
DBLK = 2048
NDB = DFF // DBLK

NC, NS = 2, 16
NW = NC * NS

GBLK = 1024


def _gate_body(x_ref, wg_ref, idx_ref, w_ref):
    logits = lax.dot_general(x_ref[...], wg_ref[...],
                             (((1,), (1,)), ((), ())),
                             preferred_element_type=jnp.float32)
    neg = jnp.full((GBLK, 1), -jnp.inf, jnp.float32)
    m1, i1 = neg, jnp.zeros((GBLK, 1), jnp.int32)
    for e in range(E):
        v = logits[:, e:e + 1]
        upd = v > m1
        m1 = jnp.where(upd, v, m1)
        i1 = jnp.where(upd, e, i1)
    m2, i2 = neg, jnp.zeros((GBLK, 1), jnp.int32)
    for e in range(E):
        v = logits[:, e:e + 1]
        upd = jnp.logical_and(v > m2, i1 != e)
        m2 = jnp.where(upd, v, m2)
        i2 = jnp.where(upd, e, i2)
    e2 = jnp.exp(m2 - m1)
    w1 = 1.0 / (1.0 + e2)
    idx_ref[...] = jnp.concatenate([i1, i2], axis=1)
    w_ref[...] = jnp.concatenate([w1, 1.0 - w1], axis=1)


def _gate(x2d, Wg):
    return pl.pallas_call(
        _gate_body,
        grid=(T // GBLK,),
        in_specs=[
            pl.BlockSpec((GBLK, D), lambda i: (i, 0)),
            pl.BlockSpec((E, D), lambda i: (0, 0)),
        ],
        out_specs=[
            pl.BlockSpec((GBLK, K), lambda i: (i, 0)),
            pl.BlockSpec((GBLK, K), lambda i: (i, 0)),
        ],
        out_shape=[
            jax.ShapeDtypeStruct((T, K), jnp.int32),
            jax.ShapeDtypeStruct((T, K), jnp.float32),
        ],
    )(x2d, Wg)


def _dispatch(topi, topw):
    ids = topi.reshape(-1)
    wts = topw.reshape(-1)
    onehot = (ids[:, None] == jnp.arange(E, dtype=jnp.int32)[None, :])
    csum = jnp.cumsum(onehot.astype(jnp.int32), axis=0)
    counts = csum[A - 1]
    pc = ((counts + TILE - 1) // TILE) * TILE
    pcc = jnp.cumsum(pc)
    poff = pcc - pc
    rank = jnp.take_along_axis(csum, ids[:, None], axis=1)[:, 0] - 1
    dest = jnp.take(poff, ids) + rank
    ar = jnp.arange(A, dtype=jnp.int32)
    gidx = (jnp.arange(PAD_N, dtype=jnp.int32) % T).at[dest].set(ar // K)
    gw = jnp.zeros((PAD_N,), jnp.float32).at[dest].set(wts)
    p0, p1 = dest[0::2], dest[1::2]
    tile_start = jnp.arange(NT, dtype=jnp.int32) * TILE
    tile_e = jnp.minimum(
        jnp.searchsorted(pcc, tile_start, side="right").astype(jnp.int32),
        E - 1)
    tile_act = (tile_start < pcc[E - 1]).astype(jnp.int32)
    return gidx, gw, p0, p1, tile_e, tile_act


_G_CHUNK = 32


def _gather_body(n_rows, tab_hbm, idx_hbm, out_hbm,
                 i0, i1, r0, r1, sem):
    per_w = n_rows // NW
    nch = per_w // _G_CHUNK
    wid = lax.axis_index("s") * NC + lax.axis_index("c")
    base = wid * per_w
    bufs = ((i0, r0), (i1, r1))

    pltpu.sync_copy(idx_hbm.at[pl.ds(base, _G_CHUNK)], i0)
    cp = pltpu.async_copy(tab_hbm.at[i0], r0, sem)
    for c in range(nch):
        _, rb = bufs[c % 2]
        if c + 1 < nch:
            inx, rnx = bufs[(c + 1) % 2]
            pltpu.sync_copy(
                idx_hbm.at[pl.ds(base + (c + 1) * _G_CHUNK, _G_CHUNK)], inx)
            cpn = pltpu.async_copy(tab_hbm.at[inx], rnx, sem)
        cp.wait()
        pltpu.sync_copy(rb, out_hbm.at[pl.ds(base + c * _G_CHUNK, _G_CHUNK)])
        if c + 1 < nch:
            cp = cpn


@functools.cache
def _sc_gather_kernel(n_rows):
    return pl.kernel(
        functools.partial(_gather_body, n_rows),
        name=f"sc_row_gather_{n_rows}",
        out_type=jax.ShapeDtypeStruct((n_rows, D), jnp.float32),
        mesh=plsc.VectorSubcoreMesh(core_axis_name="c", subcore_axis_name="s",
                                    num_cores=NC, num_subcores=NS),
        scratch_types=[
            pltpu.VMEM((_G_CHUNK,), jnp.int32),
            pltpu.VMEM((_G_CHUNK,), jnp.int32),
            pltpu.VMEM((_G_CHUNK, D), jnp.float32),
            pltpu.VMEM((_G_CHUNK, D), jnp.float32),
            pltpu.SemaphoreType.DMA,
        ],
    )


def _sc_gather(tab, idx, n_rows):
    return _sc_gather_kernel(n_rows)(tab, idx)


def _ffn_body(te_ref, ta_ref, xg_ref, w1_ref, w2_ref, w3_ref,
              b1_ref, b2_ref, b3_ref, gw_ref, out_ref, acc_ref):
    i = pl.program_id(0)
    j = pl.program_id(1)
    act = ta_ref[i] == 1

    @pl.when(act)
    def _():
        x = xg_ref[...].astype(jnp.bfloat16)
        a = lax.dot_general(x, w1_ref[0], (((1,), (1,)), ((), ())),
                            preferred_element_type=jnp.float32) + b1_ref[0]
        b = lax.dot_general(x, w2_ref[0], (((1,), (1,)), ((), ())),
                            preferred_element_type=jnp.float32) + b2_ref[0]
        h = (a * jax.nn.sigmoid(a) * b).astype(jnp.bfloat16)
        y = lax.dot_general(h, w3_ref[0], (((1,), (1,)), ((), ())),
                            preferred_element_type=jnp.float32)

        @pl.when(j == 0)
        def _():
            acc_ref[...] = y

        @pl.when(j > 0)
        def _():
            acc_ref[...] += y

    @pl.when(j == NDB - 1)
    def _():
        @pl.when(act)
        def _():
            out_ref[...] = (acc_ref[...] + b3_ref[0]) * gw_ref[0]

        @pl.when(jnp.logical_not(act))
        def _():
            out_ref[...] = jnp.zeros_like(out_ref)


def _ffn(xg, W1b, W2b, W3b, b1r, b2r, b3r, gw3, tile_e, tile_act):
    grid_spec = pltpu.PrefetchScalarGridSpec(
        num_scalar_prefetch=2,
        grid=(NT, NDB),
        in_specs=[
            pl.BlockSpec((TILE, D), lambda i, j, te, ta: (i, 0)),
            pl.BlockSpec((1, DBLK, D), lambda i, j, te, ta: (te[i], j, 0)),
            pl.BlockSpec((1, DBLK, D), lambda i, j, te, ta: (te[i], j, 0)),
            pl.BlockSpec((1, D, DBLK), lambda i, j, te, ta: (te[i], 0, j)),
            pl.BlockSpec((1, 1, DBLK), lambda i, j, te, ta: (te[i], 0, j)),
            pl.BlockSpec((1, 1, DBLK), lambda i, j, te, ta: (te[i], 0, j)),
            pl.BlockSpec((1, 1, D), lambda i, j, te, ta: (te[i], 0, 0)),
            pl.BlockSpec((1, TILE, 1), lambda i, j, te, ta: (i, 0, 0)),
        ],
        out_specs=pl.BlockSpec((TILE, D), lambda i, j, te, ta: (i, 0)),
        scratch_shapes=[pltpu.VMEM((TILE, D), jnp.float32)],
    )
    return pl.pallas_call(
        _ffn_body,
        grid_spec=grid_spec,
        out_shape=jax.ShapeDtypeStruct((PAD_N, D), jnp.float32),
    )(tile_e, tile_act, xg, W1b, W2b, W3b, b1r, b2r, b3r, gw3)


_ADD_BLK = 512


def _add_body(a_ref, b_ref, o_ref):
    o_ref[...] = a_ref[...] + b_ref[...]


def _pair_add(zz):
    return pl.pallas_call(
        _add_body,
        grid=(T // _ADD_BLK,),
        in_specs=[
            pl.BlockSpec((_ADD_BLK, D), lambda i: (i, 0)),
            pl.BlockSpec((_ADD_BLK, D), lambda i: (T // _ADD_BLK + i, 0)),
        ],
        out_specs=pl.BlockSpec((_ADD_BLK, D), lambda i: (i, 0)),
        out_shape=jax.ShapeDtypeStruct((T, D), jnp.float32),
    )(zz, zz)


def kernel(x, W1, b1, W2, b2, W3, b3, Wg):
    x2d = x.reshape(T, D)
    topi, topw = _gate(x2d, Wg)
    gidx, gw, p0, p1, tile_e, tile_act = _dispatch(topi, topw)
    xg = _sc_gather(x2d, gidx, PAD_N)
    yw = _ffn(xg,
              W1.astype(jnp.bfloat16),
              W2.astype(jnp.bfloat16),
              W3.astype(jnp.bfloat16),
              b1.reshape(E, 1, DFF), b2.reshape(E, 1, DFF),
              b3.reshape(E, 1, D),
              gw.reshape(NT, TILE, 1), tile_e, tile_act)
    zz = _sc_gather(yw, jnp.concatenate([p0, p1]), 2 * T)
    return _pair_add(zz).reshape(B, S, D)
